# baseline (speedup 1.0000x reference)
import functools

import jax
import jax.numpy as jnp
import numpy as np
from jax import lax
from jax.experimental import pallas as pl
from jax.experimental.pallas import tpu as pltpu
from jax.experimental.pallas import tpu_sc as plsc

_VOCAB = 100000
_EMBED = 128
_MAXLEN = 512
_B = 1024
_L = 200

_NC = 2
_NS = 16
_NW = _NC * _NS
_ROWS = _B * _L
_RPW = _ROWS // _NW
_CHUNK = 200
_NCHUNK = _RPW // _CHUNK
_NBUF = 4
_NROUND = _NCHUNK // _NBUF


def _pe_table() -> jnp.ndarray:
    pe = np.zeros((_MAXLEN, _EMBED), dtype=np.float32)
    position = np.arange(0, _MAXLEN)[:, np.newaxis]
    div_term = np.exp(np.arange(0, _EMBED, 2) * -(np.log(10000.0) / _EMBED))
    pe[:, 0::2] = np.sin(position * div_term)
    pe[:, 1::2] = np.cos(position * div_term)
    return jnp.asarray(pe[:_L])


_MESH = plsc.VectorSubcoreMesh(core_axis_name="c", subcore_axis_name="s")


@functools.partial(
    pl.kernel,
    out_type=jax.ShapeDtypeStruct((_ROWS, _EMBED), jnp.float32),
    mesh=_MESH,
    scratch_types=[
        pltpu.VMEM((_RPW,), jnp.int32),
        pltpu.VMEM((_NBUF, _CHUNK, _EMBED), jnp.float32),
        pltpu.VMEM_SHARED((_L, _EMBED), jnp.float32),
        [pltpu.SemaphoreType.DMA] * _NBUF,
        [pltpu.SemaphoreType.DMA] * _NBUF,
    ],
)
def _embed_lookup(x_hbm, w_hbm, pe_hbm, out_hbm, idx_v, buf, pe_sh,
                  gsems, wsems):
    wid = lax.axis_index("s") * _NC + lax.axis_index("c")
    base = wid * _RPW

    @pl.when(lax.axis_index("s") == 0)
    def _stage_pe():
        pltpu.sync_copy(pe_hbm, pe_sh)

    pltpu.sync_copy(x_hbm.at[pl.ds(base, _RPW)], idx_v)
    plsc.subcore_barrier()

    def idx_at(k):
        return idx_v.at[pl.ds(k * _CHUNK, _CHUNK)]

    for b in range(_NBUF):
        pltpu.sync_copy(pe_sh, buf.at[b])
        pltpu.async_copy(w_hbm.at[idx_at(b)], buf.at[b], gsems[b], add=True)

    def round_body(t, carry):
        for b in range(_NBUF):
            k = _NBUF * t + b
            rbase = base + k * _CHUNK
            pltpu.make_async_copy(w_hbm.at[idx_at(k)], buf.at[b],
                                  gsems[b]).wait()
            pltpu.async_copy(buf.at[b], out_hbm.at[pl.ds(rbase, _CHUNK)],
                             wsems[b])

            bj = (b - 2) % _NBUF
            j = k - 2

            @pl.when((k > 1) & (j + _NBUF < _NCHUNK))
            def _recycle_prev():
                jbase = base + j * _CHUNK
                pltpu.make_async_copy(buf.at[bj],
                                      out_hbm.at[pl.ds(jbase, _CHUNK)],
                                      wsems[bj]).wait()
                pltpu.sync_copy(pe_sh, buf.at[bj])
                pltpu.async_copy(w_hbm.at[idx_at(j + _NBUF)], buf.at[bj],
                                 gsems[bj], add=True)
        return carry

    lax.fori_loop(0, _NROUND, round_body, 0)
    for b in range(_NBUF):
        last = base + (_NCHUNK - _NBUF + b) * _CHUNK
        pltpu.make_async_copy(buf.at[b], out_hbm.at[pl.ds(last, _CHUNK)],
                              wsems[b]).wait()


def kernel(x, W):
    out = _embed_lookup(x.reshape(-1), W, _pe_table())
    return out.reshape(_B, _L, _EMBED)


# speedup vs baseline: 1.0049x; 1.0049x over previous
import functools

import jax
import jax.numpy as jnp
import numpy as np
from jax import lax
from jax.experimental import pallas as pl
from jax.experimental.pallas import tpu as pltpu
from jax.experimental.pallas import tpu_sc as plsc

_VOCAB = 100000
_EMBED = 128
_MAXLEN = 512
_B = 1024
_L = 200

_NC = 2
_NS = 16
_NW = _NC * _NS
_ROWS = _B * _L
_RPW = _ROWS // _NW
_CHUNK = 200
_NCHUNK = _RPW // _CHUNK
_NBUF = 4
_NROUND = _NCHUNK // _NBUF


def _pe_table() -> jnp.ndarray:
    pe = np.zeros((_MAXLEN, _EMBED), dtype=np.float32)
    position = np.arange(0, _MAXLEN)[:, np.newaxis]
    div_term = np.exp(np.arange(0, _EMBED, 2) * -(np.log(10000.0) / _EMBED))
    pe[:, 0::2] = np.sin(position * div_term)
    pe[:, 1::2] = np.cos(position * div_term)
    return jnp.asarray(pe[:_L])


_MESH = plsc.VectorSubcoreMesh(core_axis_name="c", subcore_axis_name="s")


@functools.partial(
    pl.kernel,
    out_type=jax.ShapeDtypeStruct((_ROWS, _EMBED), jnp.float32),
    mesh=_MESH,
    scratch_types=[
        pltpu.VMEM((_RPW,), jnp.int32),
        pltpu.VMEM((_NBUF, _CHUNK, _EMBED), jnp.float32),
        pltpu.VMEM_SHARED((_L, _EMBED), jnp.float32),
        [pltpu.SemaphoreType.DMA] * _NBUF,
        [pltpu.SemaphoreType.DMA] * _NBUF,
    ],
)
def _embed_lookup(x_hbm, w_hbm, pe_hbm, out_hbm, idx_v, buf, pe_sh,
                  gsems, wsems):
    wid = lax.axis_index("s") * _NC + lax.axis_index("c")
    base = wid * _RPW

    @pl.when(lax.axis_index("s") == 0)
    def _stage_pe():
        pltpu.sync_copy(pe_hbm, pe_sh)

    pltpu.sync_copy(x_hbm.at[pl.ds(base, _RPW)], idx_v)
    plsc.subcore_barrier()

    def idx_at(k):
        return idx_v.at[pl.ds(k * _CHUNK, _CHUNK)]

    for b in range(_NBUF):
        pltpu.sync_copy(pe_sh, buf.at[b])
        pltpu.async_copy(w_hbm.at[idx_at(b)], buf.at[b], gsems[b], add=True)

    def round_body(t, carry):
        for b in range(_NBUF):
            k = _NBUF * t + b
            rbase = base + k * _CHUNK
            pltpu.make_async_copy(w_hbm.at[idx_at(k)], buf.at[b],
                                  gsems[b]).wait()
            pltpu.async_copy(buf.at[b], out_hbm.at[pl.ds(rbase, _CHUNK)],
                             wsems[b])

            bj = (b - 1) % _NBUF
            j = k - 1

            @pl.when((k > 0) & (j + _NBUF < _NCHUNK))
            def _recycle_prev():
                jbase = base + j * _CHUNK
                pltpu.make_async_copy(buf.at[bj],
                                      out_hbm.at[pl.ds(jbase, _CHUNK)],
                                      wsems[bj]).wait()
                pltpu.sync_copy(pe_sh, buf.at[bj])
                pltpu.async_copy(w_hbm.at[idx_at(j + _NBUF)], buf.at[bj],
                                 gsems[bj], add=True)
        return carry

    lax.fori_loop(0, _NROUND, round_body, 0)
    for b in range(_NBUF):
        last = base + (_NCHUNK - _NBUF + b) * _CHUNK
        pltpu.make_async_copy(buf.at[b], out_hbm.at[pl.ds(last, _CHUNK)],
                              wsems[b]).wait()


def kernel(x, W):
    out = _embed_lookup(x.reshape(-1), W, _pe_table())
    return out.reshape(_B, _L, _EMBED)
